# baseline (speedup 1.0000x reference)
import functools

import jax
import jax.numpy as jnp
from jax import lax
from jax.experimental import pallas as pl
from jax.experimental.pallas import tpu as pltpu
from jax.experimental.pallas import tpu_sc as plsc

B = 1024
NUM = 13
NCAT = 26
V = 1000
D = 16
R1 = 64


def _stage1_body(x_ref, w_ref, idx_ref, wfm_ref):
    pos = lax.broadcasted_iota(jnp.int32, (1, V), 1).astype(jnp.float32)
    for j in range(NCAT):
        xb = x_ref[:, j * V:(j + 1) * V]
        lab = jnp.sum(xb * pos, axis=1, keepdims=True)
        idx_ref[:, j:j + 1] = (lab + 0.5).astype(jnp.int32) + j * V
        wfm_ref[:, j:j + 1] = jnp.sum(xb * w_ref[j:j + 1, :], axis=1,
                                      keepdims=True)


def _stage1(x, wfm_rows):
    return pl.pallas_call(
        _stage1_body,
        grid=(B // R1,),
        in_specs=[
            pl.BlockSpec((R1, NCAT * V), lambda i: (i, 0)),
            pl.BlockSpec((NCAT, V), lambda i: (0, 0)),
        ],
        out_specs=[
            pl.BlockSpec((R1, NCAT), lambda i: (i, 0)),
            pl.BlockSpec((R1, NCAT), lambda i: (i, 0)),
        ],
        out_shape=[
            jax.ShapeDtypeStruct((B, NCAT), jnp.int32),
            jax.ShapeDtypeStruct((B, NCAT), jnp.float32),
        ],
    )(x, wfm_rows)


def _sc_gather(table, idx):
    info = plsc.get_sparse_core_info()
    nw = info.num_cores * info.num_subcores
    n = idx.shape[0]
    b_per_w = n // nw
    mesh = plsc.VectorSubcoreMesh(core_axis_name="c", subcore_axis_name="s")

    @functools.partial(
        pl.kernel, mesh=mesh,
        compiler_params=pltpu.CompilerParams(use_tc_tiling_on_sc=False),
        out_type=jax.ShapeDtypeStruct((n, D), jnp.float32),
        scratch_types=[
            pltpu.VMEM((b_per_w,), jnp.int32),
            pltpu.VMEM((b_per_w, D), jnp.float32),
            pltpu.SemaphoreType.DMA,
        ],
    )
    def k(table_hbm, idx_hbm, out_hbm, idx_v, rows_v, sem):
        wid = lax.axis_index("s") * info.num_cores + lax.axis_index("c")
        base = wid * b_per_w
        pltpu.sync_copy(idx_hbm.at[pl.ds(base, b_per_w)], idx_v)
        pltpu.async_copy(table_hbm.at[idx_v], rows_v, sem).wait()
        pltpu.sync_copy(rows_v, out_hbm.at[pl.ds(base, b_per_w)])

    return k(table, idx)


def _stage3_body(num_ref, cat_ref, wfmv_ref, wnum_ref, ncst_ref, s_ref, a_ref,
                 w1c_ref, b1_ref, w2_ref, b2_ref, w3_ref, b3_ref, wfm_ref,
                 bfm_ref, out_ref):
    hp = lax.Precision.HIGHEST
    num = num_ref[...]
    cat = cat_ref[...]
    nsum = jnp.dot(num, wnum_ref[...], precision=hp) + ncst_ref[...]
    csum = jnp.dot(cat, s_ref[...], precision=hp)
    yfme = jnp.sum(nsum * csum, axis=1, keepdims=True)
    yfms = (jnp.dot(num, wfm_ref[...], precision=hp)
            + jnp.sum(wfmv_ref[...], axis=1, keepdims=True) + bfm_ref[...])
    h1 = jnp.maximum(
        jnp.dot(num, a_ref[...], precision=hp)
        + jnp.dot(cat, w1c_ref[...], precision=hp) + b1_ref[...], 0.0)
    h2 = jnp.maximum(jnp.dot(h1, w2_ref[...], precision=hp) + b2_ref[...], 0.0)
    yd = jnp.dot(h2, w3_ref[...], precision=hp) + b3_ref[...]
    out_ref[...] = jax.nn.sigmoid(yfme + yfms + yd)


def _stage3(num, catf, wfmv, wnum, ncst, s, a, w1c, b1c, w2, b2, w3, b3,
            wfm13, bfm):
    return pl.pallas_call(
        _stage3_body,
        out_shape=jax.ShapeDtypeStruct((B, 1), jnp.float32),
    )(num, catf, wfmv, wnum, ncst, s, a, w1c, b1c, w2, b2, w3, b3,
      wfm13, bfm)


def kernel(numeric_feats, categorical_feats, W_num, b_num, emb_tables,
           W_fm, b_fm, W1, b1, W2, b2, W3, b3):
    f32 = jnp.float32
    wfm_rows = W_fm[NUM:, 0].reshape(NCAT, V)

    a = jnp.einsum('id,idm->im', W_num, W1[:NUM * D].reshape(NUM, D, -1))
    b1c = (b1 + b_num.reshape(-1) @ W1[:NUM * D])[None]
    ncst = b_num.sum(0)[None]
    s = jnp.tile(jnp.eye(D, dtype=f32), (NCAT, 1))

    idx2, wfmv = _stage1(categorical_feats, wfm_rows)

    cat_rows = _sc_gather(emb_tables.reshape(NCAT * V, D), idx2.reshape(-1))
    catf = cat_rows.reshape(B, NCAT * D)

    return _stage3(numeric_feats, catf, wfmv,
                   W_num, ncst, s, a,
                   W1[NUM * D:], b1c, W2, b2[None], W3, b3[None],
                   W_fm[:NUM], b_fm[None])


# speedup vs baseline: 1.4757x; 1.4757x over previous
import functools

import jax
import jax.numpy as jnp
from jax import lax
from jax.experimental import pallas as pl
from jax.experimental.pallas import tpu as pltpu
from jax.experimental.pallas import tpu_sc as plsc

B = 1024
NUM = 13
NCAT = 26
V = 1000
D = 16
RB = 128


def _stage1_body(x_ref, m_ref, idx_ref):
    y = jnp.dot(x_ref[...].astype(jnp.bfloat16), m_ref[...],
                preferred_element_type=jnp.float32)
    lab = 32.0 * y[:, :NCAT] + y[:, 32:32 + NCAT] + 0.5
    offs = lax.broadcasted_iota(jnp.int32, (1, NCAT), 1) * V
    idx_ref[...] = lab.astype(jnp.int32) + offs


def _stage1(x, m):
    return pl.pallas_call(
        _stage1_body,
        grid=(B // RB,),
        in_specs=[
            pl.BlockSpec((RB, NCAT * V), lambda i: (i, 0)),
            pl.BlockSpec((NCAT * V, 64), lambda i: (0, 0)),
        ],
        out_specs=pl.BlockSpec((RB, NCAT), lambda i: (i, 0)),
        out_shape=jax.ShapeDtypeStruct((B, NCAT), jnp.int32),
    )(x, m)


def _stage1_matrix():
    p = jnp.arange(V, dtype=jnp.int32)
    hi = (p // 32).astype(jnp.float32)
    lo = (p % 32).astype(jnp.float32)
    m = jnp.zeros((NCAT, V, 64), jnp.float32)
    f = jnp.arange(NCAT)
    m = m.at[f, :, f].set(jnp.broadcast_to(hi, (NCAT, V)))
    m = m.at[f, :, 32 + f].set(jnp.broadcast_to(lo, (NCAT, V)))
    return m.reshape(NCAT * V, 64).astype(jnp.bfloat16)


def _sc_gather(table, wfm, idx):
    info = plsc.get_sparse_core_info()
    nw = info.num_cores * info.num_subcores
    n = idx.shape[0]
    b_per_w = n // nw
    mesh = plsc.VectorSubcoreMesh(core_axis_name="c", subcore_axis_name="s")

    @functools.partial(
        pl.kernel, mesh=mesh,
        compiler_params=pltpu.CompilerParams(use_tc_tiling_on_sc=False),
        out_type=[
            jax.ShapeDtypeStruct((n, D), jnp.float32),
            jax.ShapeDtypeStruct((n,), jnp.float32),
        ],
        scratch_types=[
            pltpu.VMEM((b_per_w,), jnp.int32),
            pltpu.VMEM((b_per_w, D), jnp.float32),
            pltpu.VMEM((b_per_w,), jnp.float32),
            pltpu.SemaphoreType.DMA,
            pltpu.SemaphoreType.DMA,
        ],
    )
    def k(table_hbm, wfm_hbm, idx_hbm, out_hbm, wout_hbm,
          idx_v, rows_v, wv_v, sem, sem2):
        wid = lax.axis_index("s") * info.num_cores + lax.axis_index("c")
        base = wid * b_per_w
        pltpu.sync_copy(idx_hbm.at[pl.ds(base, b_per_w)], idx_v)
        cp1 = pltpu.async_copy(table_hbm.at[idx_v], rows_v, sem)
        cp2 = pltpu.async_copy(wfm_hbm.at[idx_v], wv_v, sem2)
        cp1.wait()
        pltpu.sync_copy(rows_v, out_hbm.at[pl.ds(base, b_per_w)])
        cp2.wait()
        pltpu.sync_copy(wv_v, wout_hbm.at[pl.ds(base, b_per_w)])

    return k(table, wfm, idx)


def _stage3_body(num_ref, cat_ref, wfmv_ref, wnum_ref, ncst_ref, s_ref, a_ref,
                 w1c_ref, b1_ref, w2_ref, b2_ref, w3_ref, b3_ref, wfm_ref,
                 bfm_ref, out_ref):
    hp = lax.Precision.HIGHEST
    num = num_ref[...]
    cat = cat_ref[...]
    nsum = jnp.dot(num, wnum_ref[...], precision=hp) + ncst_ref[...]
    csum = jnp.dot(cat, s_ref[...], precision=hp)
    yfme = jnp.sum(nsum * csum, axis=1, keepdims=True)
    yfms = (jnp.dot(num, wfm_ref[...], precision=hp)
            + jnp.sum(wfmv_ref[...], axis=1, keepdims=True) + bfm_ref[...])
    h1 = jnp.maximum(
        jnp.dot(num, a_ref[...], precision=hp)
        + jnp.dot(cat, w1c_ref[...], precision=hp) + b1_ref[...], 0.0)
    h2 = jnp.maximum(jnp.dot(h1, w2_ref[...], precision=hp) + b2_ref[...], 0.0)
    yd = jnp.dot(h2, w3_ref[...], precision=hp) + b3_ref[...]
    out_ref[...] = jax.nn.sigmoid(yfme + yfms + yd)


def _stage3(num, catf, wfmv, wnum, ncst, s, a, w1c, b1c, w2, b2, w3, b3,
            wfm13, bfm):
    return pl.pallas_call(
        _stage3_body,
        out_shape=jax.ShapeDtypeStruct((B, 1), jnp.float32),
    )(num, catf, wfmv, wnum, ncst, s, a, w1c, b1c, w2, b2, w3, b3,
      wfm13, bfm)


def kernel(numeric_feats, categorical_feats, W_num, b_num, emb_tables,
           W_fm, b_fm, W1, b1, W2, b2, W3, b3):
    f32 = jnp.float32
    m = _stage1_matrix()
    wfm_flat = W_fm[NUM:, 0]

    a = jnp.einsum('id,idm->im', W_num, W1[:NUM * D].reshape(NUM, D, -1))
    b1c = (b1 + b_num.reshape(-1) @ W1[:NUM * D])[None]
    ncst = b_num.sum(0)[None]
    s = jnp.tile(jnp.eye(D, dtype=f32), (NCAT, 1))

    idx2 = _stage1(categorical_feats, m)

    cat_rows, wvals = _sc_gather(emb_tables.reshape(NCAT * V, D), wfm_flat,
                                 idx2.reshape(-1))
    catf = cat_rows.reshape(B, NCAT * D)
    wfmv = wvals.reshape(B, NCAT)

    return _stage3(numeric_feats, catf, wfmv,
                   W_num, ncst, s, a,
                   W1[NUM * D:], b1c, W2, b2[None], W3, b3[None],
                   W_fm[:NUM], b_fm[None])
